# baseline (speedup 1.0000x reference)
import functools
import jax
import jax.numpy as jnp
from jax import lax
from jax.experimental import pallas as pl
from jax.experimental.pallas import tpu as pltpu, tpu_sc as plsc

OH, OW, SR = 7, 7, 2
NBINS = OH * OW
K = SR * SR * 4
ROWS = NBINS * K
GROUP_BINS = 7
GROUP_ROWS = GROUP_BINS * K
NGROUPS = NBINS // GROUP_BINS
C = 256
CW = C // 2
NCHUNK = C // 16

HS = (256, 128, 64, 32)
BASES = (0, 65536, 81920, 86016)

NP_PAD = 1024
NWORKERS = 32
PPW = NP_PAD // NWORKERS
OUTP = NBINS * C
AHEAD = 4


def _build_idx_wt(proposals):
    N = proposals.shape[0]
    areas = (proposals[:, 2] - proposals[:, 0]) * (proposals[:, 3] - proposals[:, 1])
    scale = jnp.sqrt(areas)
    levels = jnp.clip(jnp.floor(jnp.log2(scale / 224.0) + 4.0).astype(jnp.int32), 2, 5)
    li = levels - 2
    Ls = jnp.asarray(HS, jnp.float32)[li]
    base = jnp.asarray(BASES, jnp.int32)[li]
    sp = 1.0 / jnp.asarray([4.0, 8.0, 16.0, 32.0], jnp.float32)[li]

    x1 = proposals[:, 0] * sp
    y1 = proposals[:, 1] * sp
    x2 = proposals[:, 2] * sp
    y2 = proposals[:, 3] * sp
    bin_w = jnp.maximum(x2 - x1, 1.0) / OW
    bin_h = jnp.maximum(y2 - y1, 1.0) / OH
    ty = (jnp.arange(OH * SR, dtype=jnp.float32) + 0.5) / SR
    ys = y1[:, None] + ty[None, :] * bin_h[:, None]
    xs = x1[:, None] + ty[None, :] * bin_w[:, None]

    def axis(ss):
        v = (ss >= -1.0) & (ss <= Ls[:, None])
        sc = jnp.clip(ss, 0.0, Ls[:, None] - 1.0)
        i0 = jnp.floor(sc).astype(jnp.int32)
        i1 = jnp.minimum(i0 + 1, Ls[:, None].astype(jnp.int32) - 1)
        frac = sc - i0.astype(jnp.float32)
        w0 = jnp.where(v, 1.0 - frac, 0.0)
        w1 = jnp.where(v, frac, 0.0)
        return i0, i1, w0, w1

    y0, y1i, wy0, wy1 = axis(ys)
    x0, x1i, wx0, wx1 = axis(xs)

    Wsi = Ls.astype(jnp.int32)
    yi = jnp.stack([y0, y0, y1i, y1i], axis=-1)
    wy = jnp.stack([wy0, wy0, wy1, wy1], axis=-1)
    xi = jnp.stack([x0, x1i, x0, x1i], axis=-1)
    wx = jnp.stack([wx0, wx1, wx0, wx1], axis=-1)
    idx = (base[:, None, None, None] + yi[:, :, None, :] * Wsi[:, None, None, None]
           + xi[:, None, :, :])
    wt = (wy[:, :, None, :] * wx[:, None, :, :]) * 0.25
    idx = idx.reshape(N, OH, SR, OW, SR, 4).transpose(0, 1, 3, 2, 4, 5)
    wt = wt.reshape(N, OH, SR, OW, SR, 4).transpose(0, 1, 3, 2, 4, 5)
    return idx.reshape(N * ROWS), wt.reshape(N * ROWS)


def _pack_table(feats):
    parts = []
    for f in feats:
        hw = f.shape[2] * f.shape[3]
        v = f[0].reshape(C // 32, 2, 16, hw).transpose(3, 0, 2, 1)
        bits = lax.bitcast_convert_type(v, jnp.uint32)
        bf = (bits + 0x7FFF + ((bits >> 16) & 1)) >> 16
        packed = bf[..., 0] | (bf[..., 1] << 16)
        parts.append(packed.reshape(hw, CW))
    return lax.bitcast_convert_type(jnp.concatenate(parts, axis=0), jnp.int32)


def _sc_body(table, idxs, wts, out,
             idx0, wt0, idx1, wt1, r0, r1, r2, r3, r4, r5, r6, out0, out1,
             s0, s1, s2, s3, s4, s5, s6, sem_iw, sem_o):
    wid = lax.axis_index("s") * 2 + lax.axis_index("c")
    base = wid * PPW
    ibufs = ((idx0, wt0), (idx1, wt1))
    obufs = (out0, out1)
    rbufs = (r0, r1, r2, r3, r4, r5, r6)
    sems = (s0, s1, s2, s3, s4, s5, s6)
    i49 = lax.iota(jnp.int32, 16) * NBINS

    def issue_gather(idx_ref, g, rb):
        return pltpu.async_copy(
            table.at[idx_ref.at[pl.ds(g * GROUP_ROWS, GROUP_ROWS)]],
            rbufs[rb], sems[rb])

    def issue_iw(p, s):
        off = pl.multiple_of(p * ROWS, 8)
        pltpu.async_copy(idxs.at[pl.ds(off, ROWS)], ibufs[s][0], sem_iw)
        pltpu.async_copy(wts.at[pl.ds(off, ROWS)], ibufs[s][1], sem_iw)

    def wait_iw(s):
        pltpu.make_async_copy(idxs.at[pl.ds(0, ROWS)], ibufs[s][0], sem_iw).wait()
        pltpu.make_async_copy(wts.at[pl.ds(0, ROWS)], ibufs[s][1], sem_iw).wait()

    def slot(i, s):
        p = base + 2 * i + s
        idxr, wtr = ibufs[s]
        outr = obufs[s]
        last = PPW // 2 - 1

        for g in range(NGROUPS):
            pltpu.make_async_copy(
                table.at[idxr.at[pl.ds(g * GROUP_ROWS, GROUP_ROWS)]],
                rbufs[g], sems[g]).wait()

            if g == 0:
                @pl.when(i > 0)
                def _():
                    pltpu.make_async_copy(
                        outr, out.at[pl.ds(pl.multiple_of(p * OUTP, 8), OUTP)],
                        sem_o).wait()

            rows = rbufs[g]

            def bin_body(b, _):
                bb = g * GROUP_BINS + b
                wv = wtr[pl.ds(bb * K, K)]
                acc = [jnp.zeros((16,), jnp.float32) for _ in range(NCHUNK)]
                for k in range(K):
                    w = jnp.full((16,), wv[k], jnp.float32)
                    for t in range(NCHUNK // 2):
                        v = plsc.bitcast(rows[b * K + k, pl.ds(t * 16, 16)],
                                         jnp.bfloat16)
                        lo, hi = plsc.unpack(v, format=plsc.PackFormat.INTERLEAVED)
                        acc[2 * t] = acc[2 * t] + w * lo
                        acc[2 * t + 1] = acc[2 * t + 1] + w * hi
                for c in range(NCHUNK):
                    plsc.store_scatter(outr, [i49 + (c * 16 * NBINS + bb)], acc[c])
                return 0

            lax.fori_loop(0, GROUP_BINS, bin_body, 0)

            tgt = g + AHEAD
            if tgt < NGROUPS:
                issue_gather(idxr, tgt, tgt)
            else:
                def cross():
                    if tgt == NGROUPS:
                        wait_iw(1 - s)
                    issue_gather(ibufs[1 - s][0], tgt - NGROUPS, tgt - NGROUPS)
                if s == 0:
                    cross()
                else:
                    @pl.when(i < last)
                    def _():
                        cross()

        pltpu.async_copy(
            outr, out.at[pl.ds(pl.multiple_of(p * OUTP, 8), OUTP)], sem_o)
        @pl.when(i < last)
        def _():
            issue_iw(p + 2, s)

    pltpu.sync_copy(idxs.at[pl.ds(pl.multiple_of(base * ROWS, 8), ROWS)], idx0)
    pltpu.sync_copy(wts.at[pl.ds(pl.multiple_of(base * ROWS, 8), ROWS)], wt0)
    for g in range(AHEAD):
        issue_gather(idx0, g, g)
    issue_iw(base + 1, 1)

    def body(i, carry):
        slot(i, 0)
        slot(i, 1)
        return 0

    lax.fori_loop(0, PPW // 2, body, 0)
    for k, s in ((PPW - 2, 0), (PPW - 1, 1)):
        pltpu.make_async_copy(
            obufs[s],
            out.at[pl.ds(pl.multiple_of((base + k) * OUTP, 8), OUTP)],
            sem_o).wait()


@jax.jit
def _run(table, idx_p, wt_p):
    mesh = plsc.VectorSubcoreMesh(core_axis_name="c", subcore_axis_name="s")
    return pl.kernel(
        _sc_body,
        out_type=jax.ShapeDtypeStruct((NP_PAD * OUTP,), jnp.float32),
        mesh=mesh,
        compiler_params=pltpu.CompilerParams(needs_layout_passes=False),
        scratch_types=[
            pltpu.VMEM((ROWS,), jnp.int32),
            pltpu.VMEM((ROWS,), jnp.float32),
            pltpu.VMEM((ROWS,), jnp.int32),
            pltpu.VMEM((ROWS,), jnp.float32),
        ] + [pltpu.VMEM((GROUP_ROWS, CW), jnp.int32)] * NGROUPS + [
            pltpu.VMEM((OUTP,), jnp.float32),
            pltpu.VMEM((OUTP,), jnp.float32),
        ] + [pltpu.SemaphoreType.DMA] * (NGROUPS + 2),
    )(table, idx_p, wt_p)


def kernel(feat_p2, feat_p3, feat_p4, feat_p5, proposals, im_h, im_w):
    N = proposals.shape[0]
    table = _pack_table((feat_p2, feat_p3, feat_p4, feat_p5))
    prop_p = jnp.zeros((NP_PAD, 4), jnp.float32).at[:N].set(proposals)
    idx_p, wt_p = _build_idx_wt(prop_p)
    out = _run(table, idx_p, wt_p)
    return out.reshape(NP_PAD, C, OH, OW)[:N]
